# baseline (speedup 1.0000x reference)
import functools

import jax
import jax.numpy as jnp
from jax.experimental import pallas as pl
from jax.experimental.pallas import tpu as pltpu

_BM = 200
_NBUF = 5
_NOUT = 2


def _main_kernel(
    adj_hbm, x_hbm, hx_hbm, cx_hbm, g_ref, wx_ref, wh_ref, gb_ref, bias_ref,
    hy_hbm, cy_hbm,
    adj_buf, xh_buf, cx_buf, sup_ref, hy_stage, cy_stage,
    adj_sem, xh_sem, cx_sem, hy_sem, cy_sem, *, h: int
):
    n = adj_hbm.shape[0]
    ns = n // _BM

    def adj_copy(s, b):
        return pltpu.make_async_copy(
            adj_hbm.at[pl.ds(s * _BM, _BM), :], adj_buf.at[b], adj_sem.at[b]
        )

    def hy_copy(s, b):
        return pltpu.make_async_copy(
            hy_stage.at[b], hy_hbm.at[pl.ds(s * _BM, _BM), :], hy_sem.at[b]
        )

    def cy_copy(s, b):
        return pltpu.make_async_copy(
            cy_stage.at[b], cy_hbm.at[pl.ds(s * _BM, _BM), :], cy_sem.at[b]
        )

    x_copy = pltpu.make_async_copy(x_hbm, xh_buf, xh_sem)
    hx_copy = pltpu.make_async_copy(hx_hbm, xh_buf, xh_sem)
    cx_copy = pltpu.make_async_copy(cx_hbm, cx_buf, cx_sem)

    x_copy.start()
    adj_copy(0, 0).start()
    cx_copy.start()
    for s in range(1, min(_NBUF - 1, ns)):
        adj_copy(s, s).start()

    x_copy.wait()
    sup_ref[...] = jnp.dot(
        xh_buf[...], g_ref[...], preferred_element_type=jnp.float32
    )
    hx_copy.start()

    for s in range(ns):
        b = s % _NBUF
        adj_copy(s, b).wait()

        nxt = s + _NBUF - 1
        if nxt < ns:
            adj_copy(nxt, nxt % _NBUF).start()

        if s == 0:
            hx_copy.wait()
            cx_copy.wait()

        rows = pl.ds(s * _BM, _BM)
        acc = jnp.dot(
            adj_buf[b], sup_ref[...], preferred_element_type=jnp.float32
        )
        xs = jnp.maximum(acc, 0.0) + bias_ref[...]
        gates = (
            jnp.dot(xs, wx_ref[...], preferred_element_type=jnp.float32)
            + jnp.dot(xh_buf[rows, :], wh_ref[...],
                      preferred_element_type=jnp.float32)
            + gb_ref[...]
        )
        ingate = jax.nn.sigmoid(gates[:, 0:h])
        forgetgate = jax.nn.sigmoid(gates[:, h:2 * h])
        cellgate = jnp.tanh(gates[:, 2 * h:3 * h])
        outgate = jax.nn.sigmoid(gates[:, 3 * h:4 * h])
        cy = cx_buf[rows, :] * forgetgate + ingate * cellgate
        hy = outgate * jnp.tanh(cy)

        bo = s % _NOUT
        if s >= _NOUT:
            hy_copy(s - _NOUT, bo).wait()
            cy_copy(s - _NOUT, bo).wait()
        hy_stage[bo] = hy
        cy_stage[bo] = cy
        hy_copy(s, bo).start()
        cy_copy(s, bo).start()

    for s in range(max(ns - _NOUT, 0), ns):
        hy_copy(s, s % _NOUT).wait()
        cy_copy(s, s % _NOUT).wait()


@jax.jit
def kernel(x, hx, cx, adj, gcn_weight, W_x2h, b_x2h, W_h2h, b_h2h, bias):
    n, d = x.shape
    h = hx.shape[1]

    wx_t = W_x2h.T
    wh_t = W_h2h.T
    gate_b = (b_x2h + b_h2h).reshape(1, 4 * h)
    bias2d = bias.reshape(1, h)

    hbm = pl.BlockSpec(memory_space=pltpu.MemorySpace.HBM)
    vmem = pl.BlockSpec(memory_space=pltpu.MemorySpace.VMEM)

    hy, cy = pl.pallas_call(
        functools.partial(_main_kernel, h=h),
        in_specs=[hbm, hbm, hbm, hbm, vmem, vmem, vmem, vmem, vmem],
        out_specs=[hbm, hbm],
        out_shape=[
            jax.ShapeDtypeStruct((n, h), jnp.float32),
            jax.ShapeDtypeStruct((n, h), jnp.float32),
        ],
        scratch_shapes=[
            pltpu.VMEM((_NBUF, _BM, n), jnp.float32),
            pltpu.VMEM((n, d), jnp.float32),
            pltpu.VMEM((n, h), jnp.float32),
            pltpu.VMEM((n, h), jnp.float32),
            pltpu.VMEM((_NOUT, _BM, h), jnp.float32),
            pltpu.VMEM((_NOUT, _BM, h), jnp.float32),
            pltpu.SemaphoreType.DMA((_NBUF,)),
            pltpu.SemaphoreType.DMA,
            pltpu.SemaphoreType.DMA,
            pltpu.SemaphoreType.DMA((_NOUT,)),
            pltpu.SemaphoreType.DMA((_NOUT,)),
        ],
    )(adj, x, hx, cx, gcn_weight, wx_t, wh_t, gate_b, bias2d)

    return (hy, cy)


# speedup vs baseline: 1.0998x; 1.0998x over previous
import functools

import jax
import jax.numpy as jnp
from jax.experimental import pallas as pl
from jax.experimental.pallas import tpu as pltpu


def _main_kernel(
    adj_ref, x_ref, g_ref, hx_ref, cx_ref, wx_ref, wh_ref, gb_ref, bias_ref,
    hy_ref, cy_ref, sup_ref, *, h: int
):
    @pl.when(pl.program_id(0) == 0)
    def _support():
        sup_ref[...] = jnp.dot(
            x_ref[...], g_ref[...], preferred_element_type=jnp.float32
        )

    acc = jnp.dot(
        adj_ref[...], sup_ref[...], preferred_element_type=jnp.float32
    )
    xs = jnp.maximum(acc, 0.0) + bias_ref[...]
    gates = (
        jnp.dot(xs, wx_ref[...], preferred_element_type=jnp.float32)
        + jnp.dot(hx_ref[...], wh_ref[...], preferred_element_type=jnp.float32)
        + gb_ref[...]
    )
    ingate = jax.nn.sigmoid(gates[:, 0:h])
    forgetgate = jax.nn.sigmoid(gates[:, h:2 * h])
    cellgate = jnp.tanh(gates[:, 2 * h:3 * h])
    outgate = jax.nn.sigmoid(gates[:, 3 * h:4 * h])
    cy = cx_ref[...] * forgetgate + ingate * cellgate
    cy_ref[...] = cy
    hy_ref[...] = outgate * jnp.tanh(cy)


@jax.jit
def kernel(x, hx, cx, adj, gcn_weight, W_x2h, b_x2h, W_h2h, b_h2h, bias):
    n, d = x.shape
    h = hx.shape[1]

    wx_t = W_x2h.T
    wh_t = W_h2h.T
    gate_b = (b_x2h + b_h2h).reshape(1, 4 * h)
    bias2d = bias.reshape(1, h)

    bm = 400
    nm = n // bm

    hy, cy = pl.pallas_call(
        functools.partial(_main_kernel, h=h),
        grid=(nm,),
        in_specs=[
            pl.BlockSpec((bm, n), lambda i: (i, 0)),
            pl.BlockSpec((n, d), lambda i: (0, 0)),
            pl.BlockSpec((d, h), lambda i: (0, 0)),
            pl.BlockSpec((bm, h), lambda i: (i, 0)),
            pl.BlockSpec((bm, h), lambda i: (i, 0)),
            pl.BlockSpec((h, 4 * h), lambda i: (0, 0)),
            pl.BlockSpec((h, 4 * h), lambda i: (0, 0)),
            pl.BlockSpec((1, 4 * h), lambda i: (0, 0)),
            pl.BlockSpec((1, h), lambda i: (0, 0)),
        ],
        out_specs=[
            pl.BlockSpec((bm, h), lambda i: (i, 0)),
            pl.BlockSpec((bm, h), lambda i: (i, 0)),
        ],
        out_shape=[
            jax.ShapeDtypeStruct((n, h), jnp.float32),
            jax.ShapeDtypeStruct((n, h), jnp.float32),
        ],
        scratch_shapes=[pltpu.VMEM((n, h), jnp.float32)],
        compiler_params=pltpu.CompilerParams(
            dimension_semantics=("arbitrary",),
        ),
    )(adj, x, gcn_weight, hx, cx, wx_t, wh_t, gate_b, bias2d)

    return (hy, cy)
